# baseline (speedup 1.0000x reference)
import functools

import jax
import jax.numpy as jnp
from jax import lax
from jax.experimental import pallas as pl
from jax.experimental.pallas import tpu as pltpu
from jax.experimental.pallas import tpu_sc as plsc

_VOCAB = 100000
_EMBED = 64
_BATCH = 16384

_info = plsc.get_sparse_core_info()
_NC, _NS = _info.num_cores, _info.num_subcores
_NW = _NC * _NS
_RPT = _EMBED // _NW
_HALF = _VOCAB // 2
_CSIZE = 6144
_CHUNKS = ((0, _CSIZE, 0), (_CSIZE, _CSIZE, _CSIZE), (2 * _CSIZE, _BATCH - 2 * _CSIZE, 0))
_UNROLL = 8


@functools.partial(
    pl.kernel,
    mesh=plsc.VectorSubcoreMesh(core_axis_name="c", subcore_axis_name="s"),
    compiler_params=pltpu.CompilerParams(needs_layout_passes=False),
    out_type=(
        jax.ShapeDtypeStruct((_EMBED, _BATCH), jnp.float32),
        jax.ShapeDtypeStruct((_EMBED, _BATCH), jnp.float32),
    ),
    scratch_types=[
        pltpu.VMEM((_HALF,), jnp.float32),
        pltpu.VMEM((_HALF,), jnp.float32),
        pltpu.VMEM((_BATCH,), jnp.int32),
        pltpu.VMEM((2 * _CSIZE,), jnp.float32),
        pltpu.SemaphoreType.DMA,
        pltpu.SemaphoreType.DMA,
        pltpu.SemaphoreType.DMA,
        pltpu.SemaphoreType.DMA,
        pltpu.SemaphoreType.DMA,
    ],
)
def _sc_rowgather(wm_hbm, wlv_hbm, idx_hbm, om_hbm, olv_hbm,
                  lo_v, hi_v, idx_v, ob_v, isem, asem, bsem, osem0, osem1):
    wid = lax.axis_index("s") * _NC + lax.axis_index("c")
    icopy = pltpu.async_copy(idx_hbm, idx_v, isem)

    rows = []
    for tbl, out, is_var in ((wm_hbm, om_hbm, False), (wlv_hbm, olv_hbm, True)):
        for r in range(_RPT):
            rows.append((tbl, out, wid * _RPT + r, is_var))

    def pass1(off, size, so):
        @plsc.parallel_loop(0, size, 16, unroll=_UNROLL)
        def body(i):
            iv = idx_v[pl.ds(off + i, 16)]
            g = plsc.load_gather(lo_v, [jnp.minimum(iv, _HALF - 1)])
            ob_v[pl.ds(so + i, 16)] = g

    def pass2(off, size, so, is_var):
        @plsc.parallel_loop(0, size, 16, unroll=_UNROLL)
        def body(i):
            iv = idx_v[pl.ds(off + i, 16)]
            gb = plsc.load_gather(hi_v, [jnp.maximum(iv - _HALF, 0)])
            g = jnp.where(iv < _HALF, ob_v[pl.ds(so + i, 16)], gb)
            if is_var:
                g = jnp.exp(g * 2.0)
            ob_v[pl.ds(so + i, 16)] = g

    tbl0, _, row0, _ = rows[0]
    dma_lo = pltpu.async_copy(tbl0.at[pl.ds(row0 * _VOCAB, _HALF)], lo_v, asem)
    dma_hi = pltpu.async_copy(tbl0.at[pl.ds(row0 * _VOCAB + _HALF, _HALF)], hi_v, bsem)
    icopy.wait()

    co = [None, None, None]
    for k, (tbl, out, row, is_var) in enumerate(rows):
        (o0, s0, b0), (o1, s1, b1), (o2, s2, b2) = _CHUNKS
        dma_lo.wait()
        if co[2] is not None:
            co[2].wait()
        pass1(o0, s0, b0)
        if co[1] is not None:
            co[1].wait()
        pass1(o1, s1, b1)
        dma_hi.wait()
        pass2(o0, s0, b0, is_var)
        co[0] = pltpu.async_copy(ob_v.at[pl.ds(b0, s0)], out.at[row, pl.ds(o0, s0)], osem0)
        pass2(o1, s1, b1, is_var)
        co[1] = pltpu.async_copy(ob_v.at[pl.ds(b1, s1)], out.at[row, pl.ds(o1, s1)], osem1)
        co[0].wait()
        pass1(o2, s2, b2)
        if k + 1 < len(rows):
            ntbl, _, nrow, _ = rows[k + 1]
            dma_lo = pltpu.async_copy(ntbl.at[pl.ds(nrow * _VOCAB, _HALF)], lo_v, asem)
        pass2(o2, s2, b2, is_var)
        co[2] = pltpu.async_copy(ob_v.at[pl.ds(b2, s2)], out.at[row, pl.ds(o2, s2)], osem0)
        if k + 1 < len(rows):
            ntbl, _, nrow, _ = rows[k + 1]
            dma_hi = pltpu.async_copy(ntbl.at[pl.ds(nrow * _VOCAB + _HALF, _HALF)], hi_v, bsem)
    co[1].wait()
    co[2].wait()


def kernel(indices, W_mean, W_log_var):
    idx = indices.astype(jnp.int32)
    gm, gv = _sc_rowgather(W_mean.reshape(-1), W_log_var.reshape(-1), idx)
    return gm.T, gv.T


# speedup vs baseline: 1.1177x; 1.1177x over previous
import functools

import jax
import jax.numpy as jnp
from jax import lax
from jax.experimental import pallas as pl
from jax.experimental.pallas import tpu as pltpu
from jax.experimental.pallas import tpu_sc as plsc

_VOCAB = 100000
_EMBED = 64
_BATCH = 16384

_info = plsc.get_sparse_core_info()
_NC, _NS = _info.num_cores, _info.num_subcores
_NW = _NC * _NS
_RPT = _EMBED // _NW
_OCHUNK = 6144
_CHUNKS = ((0, 6144), (6144, 6144), (12288, 4096))
_UNROLL = 8


@functools.partial(
    pl.kernel,
    mesh=plsc.VectorSubcoreMesh(core_axis_name="c", subcore_axis_name="s"),
    compiler_params=pltpu.CompilerParams(needs_layout_passes=False),
    out_type=(
        jax.ShapeDtypeStruct((_EMBED, _BATCH), jnp.float32),
        jax.ShapeDtypeStruct((_EMBED, _BATCH), jnp.float32),
    ),
    scratch_types=[
        pltpu.VMEM((_VOCAB,), jnp.float32),
        pltpu.VMEM((_BATCH,), jnp.int32),
        pltpu.VMEM((2 * _OCHUNK,), jnp.float32),
        pltpu.SemaphoreType.DMA,
        pltpu.SemaphoreType.DMA,
        pltpu.SemaphoreType.DMA,
    ],
)
def _sc_rowgather(wm_hbm, wlv_hbm, idx_hbm, om_hbm, olv_hbm, row_v, idx_v, ob_v, isem, rsem, osem):
    wid = lax.axis_index("s") * _NC + lax.axis_index("c")
    icopy = pltpu.async_copy(idx_hbm, idx_v, isem)
    half = _VOCAB // 2
    pending = []
    first = True
    for tbl, out, is_var in ((wm_hbm, om_hbm, False), (wlv_hbm, olv_hbm, True)):
        for r in range(_RPT):
            row = wid * _RPT + r
            base = row * _VOCAB
            c0 = pltpu.async_copy(tbl.at[pl.ds(base, half)], row_v.at[pl.ds(0, half)], rsem)
            c1 = pltpu.async_copy(
                tbl.at[pl.ds(base + half, _VOCAB - half)],
                row_v.at[pl.ds(half, _VOCAB - half)],
                rsem,
            )
            c0.wait()
            c1.wait()
            if first:
                icopy.wait()
                first = False
            for off, size in _CHUNKS:
                buf = len(pending) % 2
                if len(pending) >= 2:
                    pending[-2].wait()

                @plsc.parallel_loop(0, size, 16, unroll=_UNROLL)
                def body(i):
                    iv = idx_v[pl.ds(off + i, 16)]
                    g = plsc.load_gather(row_v, [iv])
                    if is_var:
                        g = jnp.exp(g * 2.0)
                    ob_v[pl.ds(buf * _OCHUNK + i, 16)] = g

                pending.append(
                    pltpu.async_copy(
                        ob_v.at[pl.ds(buf * _OCHUNK, size)],
                        out.at[row, pl.ds(off, size)],
                        osem,
                    )
                )
    pending[-2].wait()
    pending[-1].wait()


def kernel(indices, W_mean, W_log_var):
    idx = indices.astype(jnp.int32)
    gm, gv = _sc_rowgather(W_mean.reshape(-1), W_log_var.reshape(-1), idx)
    return gm.T, gv.T


# speedup vs baseline: 1.3145x; 1.1761x over previous
import functools

import jax
import jax.numpy as jnp
from jax import lax
from jax.experimental import pallas as pl
from jax.experimental.pallas import tpu as pltpu
from jax.experimental.pallas import tpu_sc as plsc

_VOCAB = 100000
_EMBED = 64
_BATCH = 16384

_info = plsc.get_sparse_core_info()
_NC, _NS = _info.num_cores, _info.num_subcores
_NW = _NC * _NS
_RPT = _EMBED // _NW
_HALF = _VOCAB // 2
_CSIZE = 6144
_CHUNKS = ((0, _CSIZE, 0), (_CSIZE, _CSIZE, _CSIZE), (2 * _CSIZE, _BATCH - 2 * _CSIZE, 0))
_UNROLL = 8


@functools.partial(
    pl.kernel,
    mesh=plsc.VectorSubcoreMesh(core_axis_name="c", subcore_axis_name="s"),
    compiler_params=pltpu.CompilerParams(needs_layout_passes=False),
    out_type=(
        jax.ShapeDtypeStruct((_EMBED, _BATCH), jnp.float32),
        jax.ShapeDtypeStruct((_EMBED, _BATCH), jnp.float32),
    ),
    scratch_types=[
        pltpu.VMEM((_HALF,), jnp.float32),
        pltpu.VMEM((_HALF,), jnp.float32),
        pltpu.VMEM((_BATCH,), jnp.int32),
        pltpu.VMEM((2 * _CSIZE,), jnp.float32),
        pltpu.SemaphoreType.DMA,
        pltpu.SemaphoreType.DMA,
        pltpu.SemaphoreType.DMA,
        pltpu.SemaphoreType.DMA,
        pltpu.SemaphoreType.DMA,
    ],
)
def _sc_rowgather(wm_hbm, wlv_hbm, idx_hbm, om_hbm, olv_hbm,
                  lo_v, hi_v, idx_v, ob_v, isem, asem, bsem, osem0, osem1):
    wid = lax.axis_index("s") * _NC + lax.axis_index("c")
    icopy = pltpu.async_copy(idx_hbm, idx_v, isem)

    rows = []
    for tbl, out, is_var in ((wm_hbm, om_hbm, False), (wlv_hbm, olv_hbm, True)):
        for r in range(_RPT):
            rows.append((tbl, out, wid * _RPT + r, is_var))

    def pass1(off, size, so):
        @plsc.parallel_loop(0, size, 16, unroll=_UNROLL)
        def body(i):
            iv = idx_v[pl.ds(off + i, 16)]
            g = plsc.load_gather(lo_v, [jnp.minimum(iv, _HALF - 1)])
            ob_v[pl.ds(so + i, 16)] = g

    def pass2(off, size, so, is_var):
        @plsc.parallel_loop(0, size, 16, unroll=_UNROLL)
        def body(i):
            iv = idx_v[pl.ds(off + i, 16)]
            gb = plsc.load_gather(hi_v, [jnp.maximum(iv - _HALF, 0)])
            g = jnp.where(iv < _HALF, ob_v[pl.ds(so + i, 16)], gb)
            if is_var:
                g = jnp.exp(g * 2.0)
            ob_v[pl.ds(so + i, 16)] = g

    tbl0, _, row0, _ = rows[0]
    dma_lo = pltpu.async_copy(tbl0.at[2 * row0], lo_v, asem)
    dma_hi = pltpu.async_copy(tbl0.at[2 * row0 + 1], hi_v, bsem)
    icopy.wait()

    co = [None, None, None]
    for k, (tbl, out, row, is_var) in enumerate(rows):
        (o0, s0, b0), (o1, s1, b1), (o2, s2, b2) = _CHUNKS
        dma_lo.wait()
        if co[2] is not None:
            co[2].wait()
        pass1(o0, s0, b0)
        if co[1] is not None:
            co[1].wait()
        pass1(o1, s1, b1)
        dma_hi.wait()
        pass2(o0, s0, b0, is_var)
        co[0] = pltpu.async_copy(ob_v.at[pl.ds(b0, s0)], out.at[row, pl.ds(o0, s0)], osem0)
        pass2(o1, s1, b1, is_var)
        co[1] = pltpu.async_copy(ob_v.at[pl.ds(b1, s1)], out.at[row, pl.ds(o1, s1)], osem1)
        co[0].wait()
        pass1(o2, s2, b2)
        if k + 1 < len(rows):
            ntbl, _, nrow, _ = rows[k + 1]
            dma_lo = pltpu.async_copy(ntbl.at[2 * nrow], lo_v, asem)
        pass2(o2, s2, b2, is_var)
        co[2] = pltpu.async_copy(ob_v.at[pl.ds(b2, s2)], out.at[row, pl.ds(o2, s2)], osem0)
        if k + 1 < len(rows):
            ntbl, _, nrow, _ = rows[k + 1]
            dma_hi = pltpu.async_copy(ntbl.at[2 * nrow + 1], hi_v, bsem)
    co[1].wait()
    co[2].wait()


def kernel(indices, W_mean, W_log_var):
    idx = indices.astype(jnp.int32)
    gm, gv = _sc_rowgather(
        W_mean.reshape(2 * _EMBED, _HALF),
        W_log_var.reshape(2 * _EMBED, _HALF),
        idx,
    )
    return gm.T, gv.T


# speedup vs baseline: 2.6483x; 2.0146x over previous
import functools

import jax
import jax.numpy as jnp
from jax import lax
from jax.experimental import pallas as pl
from jax.experimental.pallas import tpu as pltpu
from jax.experimental.pallas import tpu_sc as plsc

_VOCAB = 100000
_EMBED = 64
_BATCH = 16384

_info = plsc.get_sparse_core_info()
_NC, _NS = _info.num_cores, _info.num_subcores
_NW = _NC * _NS
_RPT = _EMBED // _NW
_OCHUNK = 6144
_CHUNKS = ((0, 6144), (6144, 6144), (12288, 4096))
_UNROLL = 8


@functools.partial(
    pl.kernel,
    mesh=plsc.VectorSubcoreMesh(core_axis_name="c", subcore_axis_name="s"),
    compiler_params=pltpu.CompilerParams(needs_layout_passes=False),
    out_type=(
        jax.ShapeDtypeStruct((_EMBED, _BATCH), jnp.float32),
        jax.ShapeDtypeStruct((_EMBED, _BATCH), jnp.float32),
    ),
    scratch_types=[
        pltpu.VMEM((_VOCAB,), jnp.float32),
        pltpu.VMEM((_BATCH,), jnp.int32),
        pltpu.VMEM((2 * _OCHUNK,), jnp.float32),
        pltpu.SemaphoreType.DMA,
        pltpu.SemaphoreType.DMA,
    ],
)
def _sc_rowgather(wm_hbm, wlv_hbm, idx_hbm, om_hbm, olv_hbm, row_v, idx_v, ob_v, isem, osem):
    wid = lax.axis_index("s") * _NC + lax.axis_index("c")
    icopy = pltpu.async_copy(idx_hbm, idx_v, isem)
    pending = []
    first = True
    for tbl, out, is_var in ((wm_hbm, om_hbm, False), (wlv_hbm, olv_hbm, True)):
        for r in range(_RPT):
            row = wid * _RPT + r
            pltpu.sync_copy(tbl.at[row], row_v)
            if first:
                icopy.wait()
                first = False
            for off, size in _CHUNKS:
                buf = len(pending) % 2
                if len(pending) >= 2:
                    pending[-2].wait()

                @plsc.parallel_loop(0, size, 16, unroll=_UNROLL)
                def body(i):
                    iv = idx_v[pl.ds(off + i, 16)]
                    g = plsc.load_gather(row_v, [iv])
                    if is_var:
                        g = jnp.exp(g * 2.0)
                    ob_v[pl.ds(buf * _OCHUNK + i, 16)] = g

                pending.append(
                    pltpu.async_copy(
                        ob_v.at[pl.ds(buf * _OCHUNK, size)],
                        out.at[row, pl.ds(off, size)],
                        osem,
                    )
                )
    pending[-2].wait()
    pending[-1].wait()


def kernel(indices, W_mean, W_log_var):
    idx = indices.astype(jnp.int32)
    gm, gv = _sc_rowgather(W_mean, W_log_var, idx)
    return gm.T, gv.T
